# baseline (speedup 1.0000x reference)
import functools

import jax
import jax.numpy as jnp
from jax import lax
from jax.experimental import pallas as pl
from jax.experimental.pallas import tpu as pltpu
from jax.experimental.pallas import tpu_sc as plsc

B, S, D, ROWS = 4, 8192, 1024, 130
N = B * S
NC, NS, L = 2, 16, 16
NW = NC * NS
TPW = N // NW
CHUNK = 32
NCHUNK = TPW // CHUNK

_mesh = plsc.VectorSubcoreMesh(core_axis_name="c", subcore_axis_name="s")


@functools.partial(
    pl.kernel,
    out_type=jax.ShapeDtypeStruct((N, D), jnp.float32),
    mesh=_mesh,
    scratch_types=[
        pltpu.VMEM((TPW,), jnp.int32),
        pltpu.VMEM((CHUNK, D), jnp.float32),
        pltpu.VMEM((CHUNK, D), jnp.float32),
        pltpu.SemaphoreType.DMA,
    ],
)
def _embed_add(x_hbm, ids_hbm, table_hbm, out_hbm, idx_v, xb, rows, sem):
    wid = lax.axis_index("s") * NC + lax.axis_index("c")
    base = wid * TPW
    pltpu.sync_copy(ids_hbm.at[pl.ds(base, TPW)], idx_v)

    def chunk_body(k, carry):
        tok = base + k * CHUNK
        gather = pltpu.async_copy(
            table_hbm.at[idx_v.at[pl.ds(k * CHUNK, CHUNK)]], rows, sem)
        pltpu.sync_copy(x_hbm.at[pl.ds(tok, CHUNK)], xb)
        gather.wait()

        def tok_body(t, c2):
            for c in range(D // L):
                sl = pl.ds(c * L, L)
                xb[t, sl] = xb[t, sl] + rows[t, sl]
            return c2

        lax.fori_loop(0, CHUNK, tok_body, 0)
        pltpu.sync_copy(xb, out_hbm.at[pl.ds(tok, CHUNK)])
        return carry

    lax.fori_loop(0, NCHUNK, chunk_body, 0)


def kernel(x, instrument_ids, table):
    ids = instrument_ids.reshape(-1).astype(jnp.int32)
    out = _embed_add(x.reshape(N, D), ids, table)
    return out.reshape(B, S, D)


# speedup vs baseline: 1.5200x; 1.5200x over previous
import functools

import jax
import jax.numpy as jnp
from jax import lax
from jax.experimental import pallas as pl
from jax.experimental.pallas import tpu as pltpu
from jax.experimental.pallas import tpu_sc as plsc

B, S, D, ROWS = 4, 8192, 1024, 130
N = B * S
NC, NS, L = 2, 16, 16
NW = NC * NS
TPW = N // NW
CH = 16
NCH = TPW // CH

_mesh = plsc.VectorSubcoreMesh(core_axis_name="c", subcore_axis_name="s")


@functools.partial(
    pl.kernel,
    out_type=jax.ShapeDtypeStruct((N, D), jnp.float32),
    mesh=_mesh,
    scratch_types=[
        pltpu.VMEM((TPW,), jnp.int32),
        pltpu.VMEM((CH, D), jnp.float32),
        pltpu.VMEM((CH, D), jnp.float32),
        pltpu.VMEM((CH, D), jnp.float32),
        pltpu.VMEM((CH, D), jnp.float32),
        pltpu.SemaphoreType.DMA,
        pltpu.SemaphoreType.DMA,
        pltpu.SemaphoreType.DMA,
        pltpu.SemaphoreType.DMA,
        pltpu.SemaphoreType.DMA,
        pltpu.SemaphoreType.DMA,
    ],
)
def _embed_add(x_hbm, ids_hbm, table_hbm, out_hbm, idx_v,
               xb0, xb1, rb0, rb1, sx0, sx1, sg0, sg1, so0, so1):
    wid = lax.axis_index("s") * NC + lax.axis_index("c")
    base = wid * TPW
    pltpu.sync_copy(ids_hbm.at[pl.ds(base, TPW)], idx_v)

    xbs, rbs = (xb0, xb1), (rb0, rb1)
    sxs, sgs, sos = (sx0, sx1), (sg0, sg1), (so0, so1)

    def issue(k, b):
        pltpu.async_copy(
            table_hbm.at[idx_v.at[pl.ds(k * CH, CH)]], rbs[b], sgs[b])
        pltpu.async_copy(x_hbm.at[pl.ds(base + k * CH, CH)], xbs[b], sxs[b])

    def wait_in(k, b):
        pltpu.make_async_copy(
            table_hbm.at[idx_v.at[pl.ds(k * CH, CH)]], rbs[b], sgs[b]).wait()
        pltpu.make_async_copy(
            x_hbm.at[pl.ds(base + k * CH, CH)], xbs[b], sxs[b]).wait()

    def store(k, b):
        pltpu.async_copy(xbs[b], out_hbm.at[pl.ds(base + k * CH, CH)], sos[b])

    def wait_store(k, b):
        pltpu.make_async_copy(
            xbs[b], out_hbm.at[pl.ds(base + k * CH, CH)], sos[b]).wait()

    def compute(b):
        xb, rb = xbs[b], rbs[b]

        def tok_body(t, c2):
            for c in range(D // L):
                sl = pl.ds(c * L, L)
                xb[t, sl] = xb[t, sl] + rb[t, sl]
            return c2

        lax.fori_loop(0, CH, tok_body, 0)

    issue(0, 0)

    def body(j, carry):
        for h in range(2):
            k = 2 * j + h
            kp = k + 1
            b, bp = h, 1 - h

            @pl.when(kp < NCH)
            def _():
                @pl.when(kp >= 2)
                def _():
                    wait_store(kp - 2, bp)
                issue(kp, bp)

            wait_in(k, b)
            compute(b)
            store(k, b)
        return carry

    lax.fori_loop(0, NCH // 2, body, 0)
    wait_store(NCH - 2, 0)
    wait_store(NCH - 1, 1)


def kernel(x, instrument_ids, table):
    ids = instrument_ids.reshape(-1).astype(jnp.int32)
    out = _embed_add(x.reshape(N, D), ids, table)
    return out.reshape(B, S, D)
